# Initial kernel scaffold; baseline (speedup 1.0000x reference)
import jax
import jax.numpy as jnp
from jax.experimental import pallas as pl


def kernel(features, segments):
    raise NotImplementedError("write your pallas kernel here")


# speedup vs baseline: 6.6335x; 6.6335x over previous
import functools

import jax
import jax.numpy as jnp
from jax import lax
from jax.experimental import pallas as pl
from jax.experimental.pallas import tpu as pltpu
from jax.experimental.pallas import tpu_sc as plsc

NUM_SEG = 10000
N_EDGES = 320000
D = 128

NC = 2
NS = 16
NW = NC * NS

EPW = N_EDGES // NW
B = 40
NB = EPW // B
SEG_CHUNKS = 5
CB = NB // SEG_CHUNKS


def _sc_body(feat_hbm, seg_hbm, zsum_hbm, zcnt_hbm, ones_hbm,
             psum_hbm, pcnt_hbm,
             seg_v, fbuf, ones_v, acc_sp, cnt_sp, sem0, sem1):
    cid = lax.axis_index("c")
    sid = lax.axis_index("s")
    wid = cid * NS + sid

    pltpu.sync_copy(ones_hbm, ones_v)

    @pl.when(sid == 0)
    def _():
        pltpu.sync_copy(zsum_hbm, acc_sp)

    @pl.when(sid == 1)
    def _():
        pltpu.sync_copy(zcnt_hbm, cnt_sp)

    plsc.subcore_barrier()

    base = wid * NB
    pltpu.async_copy(feat_hbm.at[base + 0], fbuf.at[0], sem0)
    pltpu.async_copy(feat_hbm.at[base + 1], fbuf.at[1], sem1)

    for chunk in range(SEG_CHUNKS):
        pltpu.sync_copy(seg_hbm.at[wid, chunk], seg_v)
        cbase = chunk * CB

        def _step(i, carry):
            j = 2 * i
            for b in range(2):
                lb = j + b
                jb = cbase + lb
                sem = sem0 if b == 0 else sem1
                pltpu.make_async_copy(feat_hbm.at[base + jb], fbuf.at[b], sem).wait()
                pltpu.sync_copy(fbuf.at[b], acc_sp.at[seg_v.at[lb]], add=True)
                pltpu.sync_copy(ones_v, cnt_sp.at[seg_v.at[lb]], add=True)

                @pl.when(jb + 2 < NB)
                def _():
                    pltpu.async_copy(feat_hbm.at[base + jb + 2], fbuf.at[b], sem)

            return carry

        lax.fori_loop(0, CB // 2, _step, 0)
    plsc.subcore_barrier()

    @pl.when(sid == 0)
    def _():
        pltpu.sync_copy(acc_sp, psum_hbm.at[cid])

    @pl.when(sid == 1)
    def _():
        pltpu.sync_copy(cnt_sp, pcnt_hbm.at[cid])


_sc_accumulate = functools.partial(
    pl.kernel,
    out_type=[
        jax.ShapeDtypeStruct((NC, NUM_SEG, D), jnp.float32),
        jax.ShapeDtypeStruct((NC, NUM_SEG), jnp.float32),
    ],
    mesh=plsc.VectorSubcoreMesh(core_axis_name="c", subcore_axis_name="s"),
    scratch_types=[
        pltpu.VMEM((CB, B), jnp.int32),
        pltpu.VMEM((2, B, D), jnp.float32),
        pltpu.VMEM((B,), jnp.float32),
        pltpu.VMEM_SHARED((NUM_SEG, D), jnp.float32),
        pltpu.VMEM_SHARED((NUM_SEG,), jnp.float32),
        pltpu.SemaphoreType.DMA,
        pltpu.SemaphoreType.DMA,
    ],
)(_sc_body)


RB = 1000


def _combine_body(ps_ref, pc_ref, o_ref):
    s = ps_ref[0] + ps_ref[1]
    c = pc_ref[0] + pc_ref[1]
    o_ref[...] = s / jnp.maximum(c, 1.0)


def _combine(psum, pcnt):
    return pl.pallas_call(
        _combine_body,
        grid=(NUM_SEG // RB,),
        in_specs=[
            pl.BlockSpec((NC, RB, D), lambda i: (0, i, 0)),
            pl.BlockSpec((NC, RB, 1), lambda i: (0, i, 0)),
        ],
        out_specs=pl.BlockSpec((RB, D), lambda i: (i, 0)),
        out_shape=jax.ShapeDtypeStruct((NUM_SEG, D), jnp.float32),
    )(psum, pcnt)


def kernel(features, segments):
    feat3 = features.reshape(NW * NB, B, D)
    seg4 = segments.reshape(NW, SEG_CHUNKS, CB, B)
    zsum = jnp.zeros((NUM_SEG, D), jnp.float32)
    zcnt = jnp.zeros((NUM_SEG,), jnp.float32)
    ones = jnp.ones((B,), jnp.float32)
    psum, pcnt = _sc_accumulate(feat3, seg4, zsum, zcnt, ones)
    return _combine(psum, pcnt[..., None])


# speedup vs baseline: 8.3975x; 1.2659x over previous
import functools

import jax
import jax.numpy as jnp
from jax import lax
from jax.experimental import pallas as pl
from jax.experimental.pallas import tpu as pltpu
from jax.experimental.pallas import tpu_sc as plsc

NUM_SEG = 10000
N_EDGES = 320000
D = 128

NC = 2
NS = 16
NW = NC * NS

EPW = N_EDGES // NW
B = 80
NB = EPW // B
SEG_CHUNKS = 5
CB = NB // SEG_CHUNKS


def _sc_body(feat_hbm, seg_hbm, zsum_hbm, zcnt_hbm, ones_hbm,
             psum_hbm, pcnt_hbm,
             seg_v, fbuf, ones_v, acc_sp, cnt_sp, sem0, sem1):
    cid = lax.axis_index("c")
    sid = lax.axis_index("s")
    wid = cid * NS + sid

    pltpu.sync_copy(ones_hbm, ones_v)

    @pl.when(sid == 0)
    def _():
        pltpu.sync_copy(zsum_hbm, acc_sp)

    @pl.when(sid == 1)
    def _():
        pltpu.sync_copy(zcnt_hbm, cnt_sp)

    plsc.subcore_barrier()

    base = wid * NB
    pltpu.async_copy(feat_hbm.at[base + 0], fbuf.at[0], sem0)
    pltpu.async_copy(feat_hbm.at[base + 1], fbuf.at[1], sem1)
    sems = (sem0, sem1)

    for chunk in range(SEG_CHUNKS):
        pltpu.sync_copy(seg_hbm.at[wid, chunk], seg_v)
        cbase = chunk * CB

        def _feature_block(lb, jb, buf):
            sem = sems[buf]
            pltpu.make_async_copy(feat_hbm.at[base + jb], fbuf.at[buf], sem).wait()
            pltpu.sync_copy(fbuf.at[buf], acc_sp.at[seg_v.at[lb]], add=True)
            pltpu.sync_copy(ones_v, cnt_sp.at[seg_v.at[lb]], add=True)

            @pl.when(jb + 2 < NB)
            def _():
                pltpu.async_copy(feat_hbm.at[base + jb + 2], fbuf.at[buf], sem)

        def _step(i, carry):
            j = 2 * i
            for b in range(2):
                _feature_block(j + b, cbase + j + b, (chunk + b) % 2)
            return carry

        lax.fori_loop(0, CB // 2, _step, 0)
        _feature_block(CB - 1, cbase + CB - 1, chunk % 2)
    plsc.subcore_barrier()

    @pl.when(sid == 0)
    def _():
        pltpu.sync_copy(acc_sp, psum_hbm.at[cid])

    @pl.when(sid == 1)
    def _():
        pltpu.sync_copy(cnt_sp, pcnt_hbm.at[cid])


_sc_accumulate = functools.partial(
    pl.kernel,
    out_type=[
        jax.ShapeDtypeStruct((NC, NUM_SEG, D), jnp.float32),
        jax.ShapeDtypeStruct((NC, NUM_SEG), jnp.float32),
    ],
    mesh=plsc.VectorSubcoreMesh(core_axis_name="c", subcore_axis_name="s"),
    scratch_types=[
        pltpu.VMEM((CB, B), jnp.int32),
        pltpu.VMEM((2, B, D), jnp.float32),
        pltpu.VMEM((B,), jnp.float32),
        pltpu.VMEM_SHARED((NUM_SEG, D), jnp.float32),
        pltpu.VMEM_SHARED((NUM_SEG,), jnp.float32),
        pltpu.SemaphoreType.DMA,
        pltpu.SemaphoreType.DMA,
    ],
)(_sc_body)


RB = 1000


def _combine_body(ps_ref, pc_ref, o_ref):
    s = ps_ref[0] + ps_ref[1]
    c = pc_ref[0] + pc_ref[1]
    o_ref[...] = s / jnp.maximum(c, 1.0)


def _combine(psum, pcnt):
    return pl.pallas_call(
        _combine_body,
        grid=(NUM_SEG // RB,),
        in_specs=[
            pl.BlockSpec((NC, RB, D), lambda i: (0, i, 0)),
            pl.BlockSpec((NC, RB, 1), lambda i: (0, i, 0)),
        ],
        out_specs=pl.BlockSpec((RB, D), lambda i: (i, 0)),
        out_shape=jax.ShapeDtypeStruct((NUM_SEG, D), jnp.float32),
    )(psum, pcnt)


def kernel(features, segments):
    feat3 = features.reshape(NW * NB, B, D)
    seg4 = segments.reshape(NW, SEG_CHUNKS, CB, B)
    zsum = jnp.zeros((NUM_SEG, D), jnp.float32)
    zcnt = jnp.zeros((NUM_SEG,), jnp.float32)
    ones = jnp.ones((B,), jnp.float32)
    psum, pcnt = _sc_accumulate(feat3, seg4, zsum, zcnt, ones)
    return _combine(psum, pcnt[..., None])


# speedup vs baseline: 8.8147x; 1.0497x over previous
import functools

import jax
import jax.numpy as jnp
from jax import lax
from jax.experimental import pallas as pl
from jax.experimental.pallas import tpu as pltpu
from jax.experimental.pallas import tpu_sc as plsc

NUM_SEG = 10000
N_EDGES = 320000
D = 128

NC = 2
NS = 16
NW = NC * NS

EPW = N_EDGES // NW
B = 80
NB = EPW // B
SEG_CHUNKS = 5
CB = NB // SEG_CHUNKS
NBUF = 3


def _sc_body(feat_hbm, seg_hbm, zsum_hbm, zcnt_hbm, ones_hbm,
             psum_hbm, pcnt_hbm,
             seg_v, fbuf, ones_v, acc_sp, cnt_sp,
             l0, l1, l2, s0, s1, s2, csem):
    lsem = (l0, l1, l2)
    ssem = (s0, s1, s2)
    cid = lax.axis_index("c")
    sid = lax.axis_index("s")
    wid = cid * NS + sid
    base = wid * NB

    for k in range(NBUF):
        pltpu.async_copy(feat_hbm.at[base + k], fbuf.at[k], lsem[k])
    pltpu.sync_copy(ones_hbm, ones_v)
    pltpu.sync_copy(seg_hbm.at[wid, 0], seg_v)

    @pl.when(sid == 0)
    def _():
        pltpu.sync_copy(zsum_hbm, acc_sp)

    @pl.when(sid == 1)
    def _():
        pltpu.sync_copy(zcnt_hbm, cnt_sp)

    plsc.subcore_barrier()

    def _wait_load(jb, buf):
        pltpu.make_async_copy(feat_hbm.at[base + jb], fbuf.at[buf], lsem[buf]).wait()

    def _scatter(jb, lb, buf):
        pltpu.async_copy(fbuf.at[buf], acc_sp.at[seg_v.at[lb]], ssem[buf],
                         add=True)
        pltpu.async_copy(ones_v, cnt_sp.at[seg_v.at[lb]], csem, add=True)

    def _drain_scatter(buf, lb):
        pltpu.make_async_copy(fbuf.at[buf], acc_sp.at[seg_v.at[lb]],
                              ssem[buf]).wait()

    def _drain_counts():
        def _w(i, carry):
            pltpu.make_async_copy(ones_v, cnt_sp.at[seg_v.at[0]], csem).wait()
            return carry

        lax.fori_loop(0, CB, _w, 0)

    for c in range(SEG_CHUNKS):
        cbase = c * CB
        if c > 0:
            _drain_scatter((c - 1) % NBUF, CB - 1)
            _drain_counts()
            pltpu.sync_copy(seg_hbm.at[wid, c], seg_v)

        _wait_load(cbase, c % NBUF)
        _scatter(cbase, 0, c % NBUF)
        if c > 0:
            pltpu.async_copy(feat_hbm.at[base + cbase + 2],
                             fbuf.at[(c + 2) % NBUF], lsem[(c + 2) % NBUF])

        def _step(i, carry):
            j = 3 * i
            for u in range(3):
                p = j + 1 + u
                jb = cbase + p
                buf = (c + 1 + u) % NBUF
                _wait_load(jb, buf)
                _scatter(jb, p, buf)
                _drain_scatter((c + u) % NBUF, p - 1)

                @pl.when(jb + 2 < NB)
                def _():
                    pltpu.async_copy(feat_hbm.at[base + jb + 2],
                                     fbuf.at[(c + u) % NBUF],
                                     lsem[(c + u) % NBUF])

            return carry

        lax.fori_loop(0, (CB - 1) // 3, _step, 0)

    _drain_scatter((SEG_CHUNKS - 1 + CB - 1) % NBUF, CB - 1)
    _drain_counts()
    plsc.subcore_barrier()

    @pl.when(sid == 0)
    def _():
        pltpu.sync_copy(acc_sp, psum_hbm.at[cid])

    @pl.when(sid == 1)
    def _():
        pltpu.sync_copy(cnt_sp, pcnt_hbm.at[cid])


_sc_accumulate = functools.partial(
    pl.kernel,
    out_type=[
        jax.ShapeDtypeStruct((NC, NUM_SEG, D), jnp.float32),
        jax.ShapeDtypeStruct((NC, NUM_SEG), jnp.float32),
    ],
    mesh=plsc.VectorSubcoreMesh(core_axis_name="c", subcore_axis_name="s"),
    scratch_types=[
        pltpu.VMEM((CB, B), jnp.int32),
        pltpu.VMEM((NBUF, B, D), jnp.float32),
        pltpu.VMEM((B,), jnp.float32),
        pltpu.VMEM_SHARED((NUM_SEG, D), jnp.float32),
        pltpu.VMEM_SHARED((NUM_SEG,), jnp.float32),
        pltpu.SemaphoreType.DMA,
        pltpu.SemaphoreType.DMA,
        pltpu.SemaphoreType.DMA,
        pltpu.SemaphoreType.DMA,
        pltpu.SemaphoreType.DMA,
        pltpu.SemaphoreType.DMA,
        pltpu.SemaphoreType.DMA,
    ],
)(_sc_body)


RB = 1000


def _combine_body(ps_ref, pc_ref, o_ref):
    s = ps_ref[0] + ps_ref[1]
    c = pc_ref[0] + pc_ref[1]
    o_ref[...] = s / jnp.maximum(c, 1.0)


def _combine(psum, pcnt):
    return pl.pallas_call(
        _combine_body,
        grid=(NUM_SEG // RB,),
        in_specs=[
            pl.BlockSpec((NC, RB, D), lambda i: (0, i, 0)),
            pl.BlockSpec((NC, RB, 1), lambda i: (0, i, 0)),
        ],
        out_specs=pl.BlockSpec((RB, D), lambda i: (i, 0)),
        out_shape=jax.ShapeDtypeStruct((NUM_SEG, D), jnp.float32),
    )(psum, pcnt)


def kernel(features, segments):
    feat3 = features.reshape(NW * NB, B, D)
    seg4 = segments.reshape(NW, SEG_CHUNKS, CB, B)
    zsum = jnp.zeros((NUM_SEG, D), jnp.float32)
    zcnt = jnp.zeros((NUM_SEG,), jnp.float32)
    ones = jnp.ones((B,), jnp.float32)
    psum, pcnt = _sc_accumulate(feat3, seg4, zsum, zcnt, ones)
    return _combine(psum, pcnt[..., None])
